# Initial kernel scaffold; baseline (speedup 1.0000x reference)
import jax
import jax.numpy as jnp
from jax.experimental import pallas as pl


def kernel(table, indices_tensor):
    raise NotImplementedError("write your pallas kernel here")


# speedup vs baseline: 6.0876x; 6.0876x over previous
import functools

import jax
import jax.numpy as jnp
from jax.experimental import pallas as pl
from jax.experimental.pallas import tpu as pltpu
from jax.experimental.pallas import tpu_sc as plsc

VOCAB = 1002
DIM = 128
WINDOW = 128


def kernel(table, indices_tensor):
    batch, seq = indices_tensor.shape
    n = batch * seq
    idx_flat = indices_tensor.reshape(1, n).astype(jnp.int32)

    mesh = plsc.VectorSubcoreMesh(core_axis_name="c", subcore_axis_name="s")

    @functools.partial(
        pl.kernel,
        out_type=jax.ShapeDtypeStruct((n, DIM), table.dtype),
        mesh=mesh,
    )
    def gather_kernel(table_hbm, idx_hbm, out_hbm):
        def body(idx_vmem, out_vmem):
            pltpu.sync_copy(table_hbm.at[idx_vmem.at[0]], out_vmem)

        pltpu.emit_pipeline(
            body,
            grid=(n // WINDOW,),
            in_specs=[pl.BlockSpec((1, WINDOW), lambda i: (0, i))],
            out_specs=[pl.BlockSpec((WINDOW, DIM), lambda i: (i, 0))],
            core_axis_name=("c", "s"),
            dimension_semantics=(pltpu.PARALLEL,),
        )(idx_hbm, out_hbm)

    out = gather_kernel(table, idx_flat)
    return out.reshape(batch, seq, DIM)


# speedup vs baseline: 14.8033x; 2.4317x over previous
import functools

import jax
import jax.numpy as jnp
from jax import lax
from jax.experimental import pallas as pl
from jax.experimental.pallas import tpu as pltpu
from jax.experimental.pallas import tpu_sc as plsc

VOCAB = 1002
DIM = 128
WINDOW = 256


def kernel(table, indices_tensor):
    batch, seq = indices_tensor.shape
    n = batch * seq
    idx2d = indices_tensor.reshape(n // 128, 128).astype(jnp.int32)

    mesh = plsc.VectorSubcoreMesh(core_axis_name="c", subcore_axis_name="s")

    @functools.partial(
        pl.kernel,
        out_type=jax.ShapeDtypeStruct((n, DIM), table.dtype),
        mesh=mesh,
        scratch_types=[pltpu.VMEM_SHARED((VOCAB, DIM), jnp.float32)],
    )
    def gather_kernel(table_hbm, idx_hbm, out_hbm, table_sh):
        @pl.when(lax.axis_index("s") == 0)
        def _():
            pltpu.sync_copy(table_hbm, table_sh)

        plsc.subcore_barrier()

        def body(idx_vmem, out_vmem):
            pltpu.sync_copy(table_sh.at[idx_vmem.at[0]], out_vmem.at[pl.ds(0, 128)])
            pltpu.sync_copy(table_sh.at[idx_vmem.at[1]], out_vmem.at[pl.ds(128, 128)])

        pltpu.emit_pipeline(
            body,
            grid=(n // WINDOW,),
            in_specs=[pl.BlockSpec((2, 128), lambda i: (i, 0))],
            out_specs=[pl.BlockSpec((WINDOW, DIM), lambda i: (i, 0))],
            core_axis_name=("c", "s"),
            dimension_semantics=(pltpu.PARALLEL,),
        )(idx_hbm, out_hbm)

    out = gather_kernel(table, idx2d)
    return out.reshape(batch, seq, DIM)


# speedup vs baseline: 15.4661x; 1.0448x over previous
import functools

import jax
import jax.numpy as jnp
from jax import lax
from jax.experimental import pallas as pl
from jax.experimental.pallas import tpu as pltpu
from jax.experimental.pallas import tpu_sc as plsc

VOCAB = 1002
DIM = 128
WINDOW = 256


def kernel(table, indices_tensor):
    batch, seq = indices_tensor.shape
    n = batch * seq
    idx2d = indices_tensor.reshape(n // 128, 128).astype(jnp.int32)

    mesh = plsc.VectorSubcoreMesh(core_axis_name="c", subcore_axis_name="s")

    @functools.partial(
        pl.kernel,
        out_type=jax.ShapeDtypeStruct((n, DIM), table.dtype),
        mesh=mesh,
        scratch_types=[
            pltpu.VMEM_SHARED((VOCAB, DIM), jnp.float32),
            pltpu.SemaphoreType.DMA,
        ],
    )
    def gather_kernel(table_hbm, idx_hbm, out_hbm, table_sh, gsem):
        @pl.when(lax.axis_index("s") == 0)
        def _():
            pltpu.sync_copy(table_hbm, table_sh)

        plsc.subcore_barrier()

        def body(idx_vmem, out_vmem):
            c1 = pltpu.async_copy(
                table_sh.at[idx_vmem.at[0]], out_vmem.at[pl.ds(0, 128)], gsem)
            c2 = pltpu.async_copy(
                table_sh.at[idx_vmem.at[1]], out_vmem.at[pl.ds(128, 128)], gsem)
            c1.wait()
            c2.wait()

        pltpu.emit_pipeline(
            body,
            grid=(n // WINDOW,),
            in_specs=[pl.BlockSpec((2, 128), lambda i: (i, 0))],
            out_specs=[pl.BlockSpec((WINDOW, DIM), lambda i: (i, 0))],
            core_axis_name=("c", "s"),
            dimension_semantics=(pltpu.PARALLEL,),
        )(idx_hbm, out_hbm)

    out = gather_kernel(table, idx2d)
    return out.reshape(batch, seq, DIM)
